# baseline (speedup 1.0000x reference)
import functools

import jax
import jax.numpy as jnp
from jax import lax
from jax.experimental import pallas as pl
from jax.experimental.pallas import tpu as pltpu
from jax.experimental.pallas import tpu_sc as plsc

ZETA = 0.95
ROWS_PER_BLOCK = 256
NC, NS, L = 2, 16, 16
NW = NC * NS
NBITS = 9
NB = 1 << NBITS
_SH1 = 32 - NBITS
_SH2 = 32 - 2 * NBITS
_INT_MIN = -(2 ** 31)


def _mlp_kernel(ctx_ref, W1a_ref, b1a_ref, W2a_ref, b2a_ref,
                W1b_ref, b1b_ref, W2b_ref, b2b_ref, q1_ref, q2_ref):
    ctx = ctx_ref[...]
    h1 = jnp.maximum(ctx @ W1a_ref[...] + b1a_ref[...], 0.0)
    q1_ref[...] = h1 @ W2a_ref[...] + b2a_ref[...]
    h2 = jnp.maximum(ctx @ W1b_ref[...] + b1b_ref[...], 0.0)
    q2_ref[...] = h2 @ W2b_ref[...] + b2b_ref[...]


def _run_mlps(context, W1a, b1a, W2a, b2a, W1b, b1b, W2b, b2b):
    batch, cdim = context.shape
    nh, nact = W2a.shape
    R = ROWS_PER_BLOCK
    row_spec = lambda w: pl.BlockSpec((R, w), lambda i: (i, 0))
    full_spec = lambda a, b: pl.BlockSpec((a, b), lambda i: (0, 0))
    return pl.pallas_call(
        _mlp_kernel,
        grid=(batch // R,),
        in_specs=[
            row_spec(cdim),
            full_spec(cdim, nh), full_spec(1, nh),
            full_spec(nh, nact), full_spec(1, nact),
            full_spec(cdim, nh), full_spec(1, nh),
            full_spec(nh, nact), full_spec(1, nact),
        ],
        out_specs=[row_spec(nact), row_spec(nact)],
        out_shape=[jax.ShapeDtypeStruct((batch, nact), jnp.float32),
                   jax.ShapeDtypeStruct((batch, nact), jnp.float32)],
    )(context, W1a, b1a.reshape(1, nh), W2a, b2a.reshape(1, nact),
      W1b, b1b.reshape(1, nh), W2b, b2b.reshape(1, nact))


def _ubits(vref, i):
    s = lax.bitcast_convert_type(vref[pl.ds(i * L, L)], jnp.int32)
    return jnp.where(s < 0, ~s, s ^ jnp.int32(_INT_MIN))


def _shrl(x, amount):
    return lax.shift_right_logical(x, jnp.full(x.shape, amount, jnp.int32))


HK = 8


def _zero_hists(hists):
    z = jnp.zeros((L,), jnp.float32)
    for c in range(NB // L):
        for h in hists:
            h[pl.ds(c * L, L)] = z


def _search_hist(hist_refs, z):
    nv = NB // L
    hs = []
    for c in range(nv):
        acc = hist_refs[0][pl.ds(c * L, L)]
        for hr in hist_refs[1:]:
            acc = acc + hr[pl.ds(c * L, L)]
        hs.append(acc)
    sums = [jnp.sum(h) for h in hs]
    prefix = [jnp.float32(0)]
    for c in range(nv):
        prefix.append(prefix[c] + sums[c])
    nfull = jnp.int32(0)
    for c in range(nv):
        nfull = nfull + jnp.where(prefix[c + 1] < z, 1, 0).astype(jnp.int32)
    hsel = jnp.zeros((L,), jnp.float32)
    runsel = jnp.float32(0)
    for c in range(nv):
        straddle = (prefix[c] < z) & (prefix[c + 1] >= z)
        hsel = jnp.where(straddle, hs[c], hsel)
        runsel = jnp.where(straddle, prefix[c], runsel)
    cs = plsc.cumsum(hsel) + runsel
    below = cs < z
    lane_cnt = jnp.sum(jnp.where(below, 1, 0))
    lane_mass = jnp.sum(jnp.where(below, hsel, 0.0))
    return nfull * L + lane_cnt, runsel + lane_mass


def _rows_quantile(v1ref, v2ref, pref, hists):
    hs1, hs2 = hists
    n_iters = v1ref.shape[0] // L

    _zero_hists(hs1 + hs2)

    @functools.partial(plsc.parallel_loop, 0, n_iters // HK)
    def pass1(j):
        for k in range(HK):
            i = j * HK + k
            p = pref[pl.ds(i * L, L)]
            ub1 = _ubits(v1ref, i)
            ub2 = _ubits(v2ref, i)
            plsc.addupdate_scatter(hs1[k], [_shrl(ub1, _SH1)], p)
            plsc.addupdate_scatter(hs2[k], [_shrl(ub2, _SH1)], p)

    b1a, mass1a = _search_hist(hs1, ZETA)
    b1b, mass1b = _search_hist(hs2, ZETA)

    _zero_hists(hs1 + hs2)

    @functools.partial(plsc.parallel_loop, 0, n_iters // HK)
    def pass2(j):
        for k in range(HK):
            i = j * HK + k
            p = pref[pl.ds(i * L, L)]
            ub1 = _ubits(v1ref, i)
            ub2 = _ubits(v2ref, i)
            m1 = _shrl(ub1, _SH1) == b1a
            m2 = _shrl(ub2, _SH1) == b1b
            plsc.addupdate_scatter(hs1[k],
                                   [_shrl(ub1, _SH2) & jnp.int32(NB - 1)],
                                   p, mask=m1)
            plsc.addupdate_scatter(hs2[k],
                                   [_shrl(ub2, _SH2) & jnp.int32(NB - 1)],
                                   p, mask=m2)

    b2a, _ = _search_hist(hs1, ZETA - mass1a)
    b2b, _ = _search_hist(hs2, ZETA - mass1b)

    def recon(bhi, blo):
        ub_ans = (bhi << _SH1) | (blo << _SH2) | jnp.int32(1 << (_SH2 - 1))
        s = jnp.where(ub_ans < 0, ub_ans ^ jnp.int32(_INT_MIN), ~ub_ans)
        return lax.bitcast_convert_type(s, jnp.float32)

    return recon(b1a, b2a), recon(b1b, b2b)


def _sc_quantiles(q1, q2, prop):
    batch, nact = q1.shape
    rw = batch // NW
    mesh = plsc.VectorSubcoreMesh(core_axis_name="c", subcore_axis_name="s")

    @functools.partial(
        pl.kernel,
        out_type=[jax.ShapeDtypeStruct((batch,), jnp.float32),
                  jax.ShapeDtypeStruct((batch,), jnp.float32)],
        mesh=mesh,
        compiler_params=pltpu.CompilerParams(needs_layout_passes=False),
        scratch_types=[
            pltpu.VMEM((nact,), jnp.float32),
            pltpu.VMEM((nact,), jnp.float32),
            pltpu.VMEM((nact,), jnp.float32),
            pltpu.VMEM((nact,), jnp.float32),
            pltpu.VMEM((nact,), jnp.float32),
            pltpu.VMEM((nact,), jnp.float32),
            *([pltpu.VMEM((NB,), jnp.float32)] * 16),
            pltpu.VMEM((rw,), jnp.float32),
            pltpu.VMEM((rw,), jnp.float32),
            pltpu.SemaphoreType.DMA((2,)),
            pltpu.SemaphoreType.DMA((2,)),
            pltpu.SemaphoreType.DMA((2,)),
        ],
    )
    def qkern(q1_hbm, q2_hbm, p_hbm, o1_hbm, o2_hbm,
              v1b0, v1b1, v2b0, v2b1, pb0, pb1,
              *rest):
        hsub = rest[:16]
        hs1, hs2 = tuple(hsub[:8]), tuple(hsub[8:])
        st1, st2, sq1, sq2, sp = rest[16:]
        v1b = (v1b0, v1b1)
        v2b = (v2b0, v2b1)
        pb = (pb0, pb1)
        wid = lax.axis_index("s") * NC + lax.axis_index("c")
        base = wid * rw
        lanes = lax.iota(jnp.int32, L)

        for b in range(2):
            pltpu.async_copy(q1_hbm.at[base + b], v1b[b], sq1.at[b])
            pltpu.async_copy(q2_hbm.at[base + b], v2b[b], sq2.at[b])
            pltpu.async_copy(p_hbm.at[base + b], pb[b], sp.at[b])

        def pair_body(r2, carry):
            res1, res2 = carry
            for b in range(2):
                row = r2 * 2 + b
                pltpu.make_async_copy(
                    q1_hbm.at[base + row], v1b[b], sq1.at[b]).wait()
                pltpu.make_async_copy(
                    q2_hbm.at[base + row], v2b[b], sq2.at[b]).wait()
                pltpu.make_async_copy(
                    p_hbm.at[base + row], pb[b], sp.at[b]).wait()

                val1, val2 = _rows_quantile(v1b[b], v2b[b], pb[b],
                                            (hs1, hs2))

                @pl.when(row < rw - 2)
                def _():
                    nxt = base + row + 2
                    pltpu.async_copy(q1_hbm.at[nxt], v1b[b], sq1.at[b])
                    pltpu.async_copy(q2_hbm.at[nxt], v2b[b], sq2.at[b])
                    pltpu.async_copy(p_hbm.at[nxt], pb[b], sp.at[b])

                lane = row % L
                res1 = jnp.where(lanes == lane, val1, res1)
                res2 = jnp.where(lanes == lane, val2, res2)
                if b == 1:
                    @pl.when(row % L == L - 1)
                    def _():
                        st1[pl.ds(row - (L - 1), L)] = res1
                        st2[pl.ds(row - (L - 1), L)] = res2
            return res1, res2

        z = jnp.zeros((L,), jnp.float32)
        lax.fori_loop(0, rw // 2, pair_body, (z, z))
        pltpu.sync_copy(st1, o1_hbm.at[pl.ds(base, rw)])
        pltpu.sync_copy(st2, o2_hbm.at[pl.ds(base, rw)])

    return qkern(q1, q2, prop)


def kernel(context, log_pi, propensity, split, W1a, b1a, W2a, b2a, W1b, b1b, W2b, b2b):
    del log_pi
    q1, q2 = _run_mlps(context, W1a, b1a, W2a, b2a, W1b, b1b, W2b, b2b)
    v1, v2 = _sc_quantiles(q1, q2, propensity)
    return (1.0 - split) * v1 + split * v2


# speedup vs baseline: 2.4402x; 2.4402x over previous
import functools

import jax
import jax.numpy as jnp
from jax import lax
from jax.experimental import pallas as pl
from jax.experimental.pallas import tpu as pltpu
from jax.experimental.pallas import tpu_sc as plsc

ZETA = 0.95
ROWS_PER_BLOCK = 256
NC, NS, L = 2, 16, 16
NW = NC * NS
NBITS = 9
NB = 1 << NBITS
_SH1 = 32 - NBITS
_SH2 = 32 - 2 * NBITS
_INT_MIN = -(2 ** 31)


def _mlp_kernel(ctx_ref, W1a_ref, b1a_ref, W2a_ref, b2a_ref,
                W1b_ref, b1b_ref, W2b_ref, b2b_ref, q1_ref, q2_ref):
    ctx = ctx_ref[...]
    h1 = jnp.maximum(ctx @ W1a_ref[...] + b1a_ref[...], 0.0)
    q1_ref[...] = h1 @ W2a_ref[...] + b2a_ref[...]
    h2 = jnp.maximum(ctx @ W1b_ref[...] + b1b_ref[...], 0.0)
    q2_ref[...] = h2 @ W2b_ref[...] + b2b_ref[...]


def _run_mlps(context, W1a, b1a, W2a, b2a, W1b, b1b, W2b, b2b, start, n):
    batch, cdim = context.shape
    nh, nact = W2a.shape
    R = ROWS_PER_BLOCK
    off = start // R
    in_row = lambda w: pl.BlockSpec((R, w), lambda i: (i + off, 0))
    out_row = lambda w: pl.BlockSpec((R, w), lambda i: (i, 0))
    full_spec = lambda a, b: pl.BlockSpec((a, b), lambda i: (0, 0))
    return pl.pallas_call(
        _mlp_kernel,
        grid=(n // R,),
        in_specs=[
            in_row(cdim),
            full_spec(cdim, nh), full_spec(1, nh),
            full_spec(nh, nact), full_spec(1, nact),
            full_spec(cdim, nh), full_spec(1, nh),
            full_spec(nh, nact), full_spec(1, nact),
        ],
        out_specs=[out_row(nact), out_row(nact)],
        out_shape=[jax.ShapeDtypeStruct((n, nact), jnp.float32),
                   jax.ShapeDtypeStruct((n, nact), jnp.float32)],
    )(context, W1a, b1a.reshape(1, nh), W2a, b2a.reshape(1, nact),
      W1b, b1b.reshape(1, nh), W2b, b2b.reshape(1, nact))


def _ubits(vref, i):
    s = lax.bitcast_convert_type(vref[pl.ds(i * L, L)], jnp.int32)
    return jnp.where(s < 0, ~s, s ^ jnp.int32(_INT_MIN))


def _shrl(x, amount):
    return lax.shift_right_logical(x, jnp.full(x.shape, amount, jnp.int32))


HK = 8


def _zero_hists(hists):
    z = jnp.zeros((L,), jnp.float32)
    for c in range(NB // L):
        for h in hists:
            h[pl.ds(c * L, L)] = z


def _search_hist(hist_refs, z):
    nv = NB // L
    hs = []
    for c in range(nv):
        acc = hist_refs[0][pl.ds(c * L, L)]
        for hr in hist_refs[1:]:
            acc = acc + hr[pl.ds(c * L, L)]
        hs.append(acc)
    sums = [jnp.sum(h) for h in hs]
    prefix = [jnp.float32(0)]
    for c in range(nv):
        prefix.append(prefix[c] + sums[c])
    nfull = jnp.int32(0)
    for c in range(nv):
        nfull = nfull + jnp.where(prefix[c + 1] < z, 1, 0).astype(jnp.int32)
    hsel = jnp.zeros((L,), jnp.float32)
    runsel = jnp.float32(0)
    for c in range(nv):
        straddle = (prefix[c] < z) & (prefix[c + 1] >= z)
        hsel = jnp.where(straddle, hs[c], hsel)
        runsel = jnp.where(straddle, prefix[c], runsel)
    cs = plsc.cumsum(hsel) + runsel
    below = cs < z
    lane_cnt = jnp.sum(jnp.where(below, 1, 0))
    lane_mass = jnp.sum(jnp.where(below, hsel, 0.0))
    return nfull * L + lane_cnt, runsel + lane_mass


def _rows_quantile(v1ref, v2ref, pref, hists):
    h1, h2 = hists
    n_iters = v1ref.shape[0] // L

    _zero_hists(hists)

    def pass1(j, carry):
        staged = []
        for k in range(HK):
            i = j * HK + k
            p = pref[pl.ds(i * L, L)]
            ub1 = _ubits(v1ref, i)
            ub2 = _ubits(v2ref, i)
            staged.append((p, _shrl(ub1, _SH1), _shrl(ub2, _SH1)))
        for p, bk1, bk2 in staged:
            plsc.addupdate_scatter(h1, [bk1], p)
            plsc.addupdate_scatter(h2, [bk2], p)
        return carry
    lax.fori_loop(0, n_iters // HK, pass1, jnp.int32(0))

    b1a, mass1a = _search_hist(hists[:1], ZETA)
    b1b, mass1b = _search_hist(hists[1:], ZETA)

    _zero_hists(hists)

    def pass2(j, carry):
        staged = []
        for k in range(HK):
            i = j * HK + k
            p = pref[pl.ds(i * L, L)]
            ub1 = _ubits(v1ref, i)
            ub2 = _ubits(v2ref, i)
            m1 = _shrl(ub1, _SH1) == b1a
            m2 = _shrl(ub2, _SH1) == b1b
            staged.append((p, _shrl(ub1, _SH2) & jnp.int32(NB - 1), m1,
                           _shrl(ub2, _SH2) & jnp.int32(NB - 1), m2))
        for p, bk1, m1, bk2, m2 in staged:
            plsc.addupdate_scatter(h1, [bk1], p, mask=m1)
            plsc.addupdate_scatter(h2, [bk2], p, mask=m2)
        return carry
    lax.fori_loop(0, n_iters // HK, pass2, jnp.int32(0))

    b2a, _ = _search_hist(hists[:1], ZETA - mass1a)
    b2b, _ = _search_hist(hists[1:], ZETA - mass1b)

    def recon(bhi, blo):
        ub_ans = (bhi << _SH1) | (blo << _SH2) | jnp.int32(1 << (_SH2 - 1))
        s = jnp.where(ub_ans < 0, ub_ans ^ jnp.int32(_INT_MIN), ~ub_ans)
        return lax.bitcast_convert_type(s, jnp.float32)

    return recon(b1a, b2a), recon(b1b, b2b)


def _sc_quantiles(q1, q2, prop, nrows):
    batch, nact = q1.shape
    rw = nrows // NW
    mesh = plsc.VectorSubcoreMesh(core_axis_name="c", subcore_axis_name="s")

    @functools.partial(
        pl.kernel,
        out_type=[jax.ShapeDtypeStruct((nrows,), jnp.float32),
                  jax.ShapeDtypeStruct((nrows,), jnp.float32)],
        mesh=mesh,
        compiler_params=pltpu.CompilerParams(needs_layout_passes=False),
        scratch_types=[
            *([pltpu.VMEM((nact,), jnp.float32)] * 9),
            pltpu.VMEM((NB,), jnp.float32),
            pltpu.VMEM((NB,), jnp.float32),
            pltpu.VMEM((((rw + L - 1) // L) * L,), jnp.float32),
            pltpu.VMEM((((rw + L - 1) // L) * L,), jnp.float32),
            pltpu.SemaphoreType.DMA((3,)),
            pltpu.SemaphoreType.DMA((3,)),
            pltpu.SemaphoreType.DMA((3,)),
        ],
    )
    def qkern(q1_hbm, q2_hbm, p_hbm, o1_hbm, o2_hbm,
              v1b0, v1b1, v1b2, v2b0, v2b1, v2b2, pb0, pb1, pb2,
              h1, h2, st1, st2, sq1, sq2, sp):
        v1b = (v1b0, v1b1, v1b2)
        v2b = (v2b0, v2b1, v2b2)
        pb = (pb0, pb1, pb2)
        wid = lax.axis_index("s") * NC + lax.axis_index("c")
        base = wid * rw
        lanes = lax.iota(jnp.int32, L)

        for b in range(2):
            pltpu.async_copy(q1_hbm.at[base + b], v1b[b], sq1.at[b])
            pltpu.async_copy(q2_hbm.at[base + b], v2b[b], sq2.at[b])
            pltpu.async_copy(p_hbm.at[base + b], pb[b], sp.at[b])

        def tri_body(r3, carry):
            res1, res2 = carry
            for b in range(3):
                row = r3 * 3 + b
                pltpu.make_async_copy(
                    q1_hbm.at[base + row], v1b[b], sq1.at[b]).wait()
                pltpu.make_async_copy(
                    q2_hbm.at[base + row], v2b[b], sq2.at[b]).wait()
                pltpu.make_async_copy(
                    p_hbm.at[base + row], pb[b], sp.at[b]).wait()

                nb = (b + 2) % 3
                @pl.when(row < rw - 2)
                def _():
                    nxt = base + row + 2
                    pltpu.async_copy(q1_hbm.at[nxt], v1b[nb], sq1.at[nb])
                    pltpu.async_copy(q2_hbm.at[nxt], v2b[nb], sq2.at[nb])
                    pltpu.async_copy(p_hbm.at[nxt], pb[nb], sp.at[nb])

                val1, val2 = _rows_quantile(v1b[b], v2b[b], pb[b], (h1, h2))

                lane = row % L
                res1 = jnp.where(lanes == lane, val1, res1)
                res2 = jnp.where(lanes == lane, val2, res2)

                @pl.when(row % L == L - 1)
                def _():
                    st1[pl.ds(row - (L - 1), L)] = res1
                    st2[pl.ds(row - (L - 1), L)] = res2
            return res1, res2

        z = jnp.zeros((L,), jnp.float32)
        res1, res2 = lax.fori_loop(0, rw // 3, tri_body, (z, z))
        if rw % L:
            st1[pl.ds(rw - rw % L, L)] = res1
            st2[pl.ds(rw - rw % L, L)] = res2
        pltpu.sync_copy(st1.at[pl.ds(0, rw)], o1_hbm.at[pl.ds(base, rw)])
        pltpu.sync_copy(st2.at[pl.ds(0, rw)], o2_hbm.at[pl.ds(base, rw)])

    return qkern(q1, q2, prop)


S_SC = 1536
TC_R = 256


def _ukey_u32(x):
    u = lax.bitcast_convert_type(x, jnp.uint32)
    sign = u >= jnp.uint32(0x80000000)
    return jnp.where(sign, ~u, u | jnp.uint32(0x80000000))


def _wquantile_block(q, p, zeta):
    uk = _ukey_u32(q)
    rows = q.shape[0]
    lo0 = jnp.zeros((rows, 1), jnp.uint32)
    hi0 = jnp.full((rows, 1), 0xFFFFFFFF, jnp.uint32)

    def body(_, carry):
        lo, hi = carry
        mid = lo + ((hi - lo) >> 1)
        g = jnp.sum(jnp.where(uk <= mid, p, 0.0), axis=1, keepdims=True)
        pred = g >= zeta
        return jnp.where(pred, lo, mid + 1), jnp.where(pred, mid, hi)

    lo, _ = lax.fori_loop(0, 18, body, (lo0, hi0))
    sign = lo >= jnp.uint32(0x80000000)
    bits = jnp.where(sign, lo ^ jnp.uint32(0x80000000), ~lo)
    return lax.bitcast_convert_type(bits, jnp.float32)


def _tc_search_kernel(q1_ref, q2_ref, prop_ref, split_ref, out_ref):
    p = prop_ref[...]
    v1 = _wquantile_block(q1_ref[...], p, ZETA)
    v2 = _wquantile_block(q2_ref[...], p, ZETA)
    s = split_ref[...]
    out_ref[...] = (1.0 - s) * v1 + s * v2


def _tc_search(q1, q2, prop, split2, start):
    batch, nact = q1.shape
    n = batch - start
    off = start // TC_R
    shifted = lambda w: pl.BlockSpec((TC_R, w), lambda i: (i + off, 0))
    return pl.pallas_call(
        _tc_search_kernel,
        grid=(n // TC_R,),
        in_specs=[shifted(nact), shifted(nact), shifted(nact), shifted(1)],
        out_specs=pl.BlockSpec((TC_R, 1), lambda i: (i, 0)),
        out_shape=jax.ShapeDtypeStruct((n, 1), jnp.float32),
    )(q1, q2, prop, split2)


def kernel(context, log_pi, propensity, split, W1a, b1a, W2a, b2a, W1b, b1b, W2b, b2b):
    del log_pi
    batch = context.shape[0]
    q1, q2 = _run_mlps(context, W1a, b1a, W2a, b2a, W1b, b1b, W2b, b2b,
                       0, batch)
    v1, v2 = _sc_quantiles(q1, q2, propensity, S_SC)
    split2 = split.reshape(batch, 1)
    b_tc = _tc_search(q1, q2, propensity, split2, S_SC)
    s_lo = split[:S_SC]
    b_sc = (1.0 - s_lo) * v1 + s_lo * v2
    return jnp.concatenate([b_sc, b_tc.reshape(batch - S_SC)])


# speedup vs baseline: 2.4404x; 1.0000x over previous
import functools

import jax
import jax.numpy as jnp
from jax import lax
from jax.experimental import pallas as pl
from jax.experimental.pallas import tpu as pltpu
from jax.experimental.pallas import tpu_sc as plsc

ZETA = 0.95
ROWS_PER_BLOCK = 256
NC, NS, L = 2, 16, 16
NW = NC * NS
NBITS = 9
NB = 1 << NBITS
_SH1 = 32 - NBITS
_SH2 = 32 - 2 * NBITS
_INT_MIN = -(2 ** 31)


def _mlp_kernel(ctx_ref, W1a_ref, b1a_ref, W2a_ref, b2a_ref,
                W1b_ref, b1b_ref, W2b_ref, b2b_ref, q1_ref, q2_ref):
    ctx = ctx_ref[...]
    h1 = jnp.maximum(ctx @ W1a_ref[...] + b1a_ref[...], 0.0)
    q1_ref[...] = h1 @ W2a_ref[...] + b2a_ref[...]
    h2 = jnp.maximum(ctx @ W1b_ref[...] + b1b_ref[...], 0.0)
    q2_ref[...] = h2 @ W2b_ref[...] + b2b_ref[...]


def _run_mlps(context, W1a, b1a, W2a, b2a, W1b, b1b, W2b, b2b, start, n):
    batch, cdim = context.shape
    nh, nact = W2a.shape
    R = ROWS_PER_BLOCK
    off = start // R
    in_row = lambda w: pl.BlockSpec((R, w), lambda i: (i + off, 0))
    out_row = lambda w: pl.BlockSpec((R, w), lambda i: (i, 0))
    full_spec = lambda a, b: pl.BlockSpec((a, b), lambda i: (0, 0))
    return pl.pallas_call(
        _mlp_kernel,
        grid=(n // R,),
        in_specs=[
            in_row(cdim),
            full_spec(cdim, nh), full_spec(1, nh),
            full_spec(nh, nact), full_spec(1, nact),
            full_spec(cdim, nh), full_spec(1, nh),
            full_spec(nh, nact), full_spec(1, nact),
        ],
        out_specs=[out_row(nact), out_row(nact)],
        out_shape=[jax.ShapeDtypeStruct((n, nact), jnp.float32),
                   jax.ShapeDtypeStruct((n, nact), jnp.float32)],
    )(context, W1a, b1a.reshape(1, nh), W2a, b2a.reshape(1, nact),
      W1b, b1b.reshape(1, nh), W2b, b2b.reshape(1, nact))


def _ubits(vref, i):
    s = lax.bitcast_convert_type(vref[pl.ds(i * L, L)], jnp.int32)
    return jnp.where(s < 0, ~s, s ^ jnp.int32(_INT_MIN))


def _shrl(x, amount):
    return lax.shift_right_logical(x, jnp.full(x.shape, amount, jnp.int32))


HK = 4


def _zero_hists(hists):
    z = jnp.zeros((L,), jnp.float32)
    for c in range(NB // L):
        for h in hists:
            h[pl.ds(c * L, L)] = z


def _search_hist(hist_refs, z):
    nv = NB // L
    hs = []
    for c in range(nv):
        acc = hist_refs[0][pl.ds(c * L, L)]
        for hr in hist_refs[1:]:
            acc = acc + hr[pl.ds(c * L, L)]
        hs.append(acc)
    sums = [jnp.sum(h) for h in hs]
    prefix = [jnp.float32(0)]
    for c in range(nv):
        prefix.append(prefix[c] + sums[c])
    nfull = jnp.int32(0)
    for c in range(nv):
        nfull = nfull + jnp.where(prefix[c + 1] < z, 1, 0).astype(jnp.int32)
    hsel = jnp.zeros((L,), jnp.float32)
    runsel = jnp.float32(0)
    for c in range(nv):
        straddle = (prefix[c] < z) & (prefix[c + 1] >= z)
        hsel = jnp.where(straddle, hs[c], hsel)
        runsel = jnp.where(straddle, prefix[c], runsel)
    cs = plsc.cumsum(hsel) + runsel
    below = cs < z
    lane_cnt = jnp.sum(jnp.where(below, 1, 0))
    lane_mass = jnp.sum(jnp.where(below, hsel, 0.0))
    return nfull * L + lane_cnt, runsel + lane_mass


def _rows_quantile(v1ref, v2ref, pref, hists):
    h1, h2 = hists
    n_iters = v1ref.shape[0] // L

    _zero_hists(hists)

    def pass1(j, carry):
        staged = []
        for k in range(HK):
            i = j * HK + k
            p = pref[pl.ds(i * L, L)]
            ub1 = _ubits(v1ref, i)
            ub2 = _ubits(v2ref, i)
            staged.append((p, _shrl(ub1, _SH1), _shrl(ub2, _SH1)))
        for p, bk1, bk2 in staged:
            plsc.addupdate_scatter(h1, [bk1], p)
            plsc.addupdate_scatter(h2, [bk2], p)
        return carry
    lax.fori_loop(0, n_iters // HK, pass1, jnp.int32(0))

    b1a, mass1a = _search_hist(hists[:1], ZETA)
    b1b, mass1b = _search_hist(hists[1:], ZETA)

    _zero_hists(hists)

    def pass2(j, carry):
        staged = []
        for k in range(HK):
            i = j * HK + k
            p = pref[pl.ds(i * L, L)]
            ub1 = _ubits(v1ref, i)
            ub2 = _ubits(v2ref, i)
            m1 = _shrl(ub1, _SH1) == b1a
            m2 = _shrl(ub2, _SH1) == b1b
            staged.append((p, _shrl(ub1, _SH2) & jnp.int32(NB - 1), m1,
                           _shrl(ub2, _SH2) & jnp.int32(NB - 1), m2))
        for p, bk1, m1, bk2, m2 in staged:
            plsc.addupdate_scatter(h1, [bk1], p, mask=m1)
            plsc.addupdate_scatter(h2, [bk2], p, mask=m2)
        return carry
    lax.fori_loop(0, n_iters // HK, pass2, jnp.int32(0))

    b2a, _ = _search_hist(hists[:1], ZETA - mass1a)
    b2b, _ = _search_hist(hists[1:], ZETA - mass1b)

    def recon(bhi, blo):
        ub_ans = (bhi << _SH1) | (blo << _SH2) | jnp.int32(1 << (_SH2 - 1))
        s = jnp.where(ub_ans < 0, ub_ans ^ jnp.int32(_INT_MIN), ~ub_ans)
        return lax.bitcast_convert_type(s, jnp.float32)

    return recon(b1a, b2a), recon(b1b, b2b)


def _sc_quantiles(q1, q2, prop, nrows):
    batch, nact = q1.shape
    rw = nrows // NW
    mesh = plsc.VectorSubcoreMesh(core_axis_name="c", subcore_axis_name="s")

    @functools.partial(
        pl.kernel,
        out_type=[jax.ShapeDtypeStruct((nrows,), jnp.float32),
                  jax.ShapeDtypeStruct((nrows,), jnp.float32)],
        mesh=mesh,
        compiler_params=pltpu.CompilerParams(needs_layout_passes=False),
        scratch_types=[
            *([pltpu.VMEM((nact,), jnp.float32)] * 9),
            pltpu.VMEM((NB,), jnp.float32),
            pltpu.VMEM((NB,), jnp.float32),
            pltpu.VMEM((((rw + L - 1) // L) * L,), jnp.float32),
            pltpu.VMEM((((rw + L - 1) // L) * L,), jnp.float32),
            pltpu.SemaphoreType.DMA((3,)),
            pltpu.SemaphoreType.DMA((3,)),
            pltpu.SemaphoreType.DMA((3,)),
        ],
    )
    def qkern(q1_hbm, q2_hbm, p_hbm, o1_hbm, o2_hbm,
              v1b0, v1b1, v1b2, v2b0, v2b1, v2b2, pb0, pb1, pb2,
              h1, h2, st1, st2, sq1, sq2, sp):
        v1b = (v1b0, v1b1, v1b2)
        v2b = (v2b0, v2b1, v2b2)
        pb = (pb0, pb1, pb2)
        wid = lax.axis_index("s") * NC + lax.axis_index("c")
        base = wid * rw
        lanes = lax.iota(jnp.int32, L)

        for b in range(2):
            pltpu.async_copy(q1_hbm.at[base + b], v1b[b], sq1.at[b])
            pltpu.async_copy(q2_hbm.at[base + b], v2b[b], sq2.at[b])
            pltpu.async_copy(p_hbm.at[base + b], pb[b], sp.at[b])

        def tri_body(r3, carry):
            res1, res2 = carry
            for b in range(3):
                row = r3 * 3 + b
                pltpu.make_async_copy(
                    q1_hbm.at[base + row], v1b[b], sq1.at[b]).wait()
                pltpu.make_async_copy(
                    q2_hbm.at[base + row], v2b[b], sq2.at[b]).wait()
                pltpu.make_async_copy(
                    p_hbm.at[base + row], pb[b], sp.at[b]).wait()

                nb = (b + 2) % 3
                @pl.when(row < rw - 2)
                def _():
                    nxt = base + row + 2
                    pltpu.async_copy(q1_hbm.at[nxt], v1b[nb], sq1.at[nb])
                    pltpu.async_copy(q2_hbm.at[nxt], v2b[nb], sq2.at[nb])
                    pltpu.async_copy(p_hbm.at[nxt], pb[nb], sp.at[nb])

                val1, val2 = _rows_quantile(v1b[b], v2b[b], pb[b], (h1, h2))

                lane = row % L
                res1 = jnp.where(lanes == lane, val1, res1)
                res2 = jnp.where(lanes == lane, val2, res2)

                @pl.when(row % L == L - 1)
                def _():
                    st1[pl.ds(row - (L - 1), L)] = res1
                    st2[pl.ds(row - (L - 1), L)] = res2
            return res1, res2

        z = jnp.zeros((L,), jnp.float32)
        res1, res2 = lax.fori_loop(0, rw // 3, tri_body, (z, z))
        if rw % L:
            st1[pl.ds(rw - rw % L, L)] = res1
            st2[pl.ds(rw - rw % L, L)] = res2
        pltpu.sync_copy(st1.at[pl.ds(0, rw)], o1_hbm.at[pl.ds(base, rw)])
        pltpu.sync_copy(st2.at[pl.ds(0, rw)], o2_hbm.at[pl.ds(base, rw)])

    return qkern(q1, q2, prop)


S_SC = 1536
TC_R = 256


def _ukey_u32(x):
    u = lax.bitcast_convert_type(x, jnp.uint32)
    sign = u >= jnp.uint32(0x80000000)
    return jnp.where(sign, ~u, u | jnp.uint32(0x80000000))


def _wquantile_block(q, p, zeta):
    uk = _ukey_u32(q)
    rows = q.shape[0]
    lo0 = jnp.zeros((rows, 1), jnp.uint32)
    hi0 = jnp.full((rows, 1), 0xFFFFFFFF, jnp.uint32)

    def body(_, carry):
        lo, hi = carry
        mid = lo + ((hi - lo) >> 1)
        g = jnp.sum(jnp.where(uk <= mid, p, 0.0), axis=1, keepdims=True)
        pred = g >= zeta
        return jnp.where(pred, lo, mid + 1), jnp.where(pred, mid, hi)

    lo, _ = lax.fori_loop(0, 18, body, (lo0, hi0))
    sign = lo >= jnp.uint32(0x80000000)
    bits = jnp.where(sign, lo ^ jnp.uint32(0x80000000), ~lo)
    return lax.bitcast_convert_type(bits, jnp.float32)


def _tc_search_kernel(q1_ref, q2_ref, prop_ref, split_ref, out_ref):
    p = prop_ref[...]
    v1 = _wquantile_block(q1_ref[...], p, ZETA)
    v2 = _wquantile_block(q2_ref[...], p, ZETA)
    s = split_ref[...]
    out_ref[...] = (1.0 - s) * v1 + s * v2


def _tc_search(q1, q2, prop, split2, start):
    batch, nact = q1.shape
    n = batch - start
    off = start // TC_R
    shifted = lambda w: pl.BlockSpec((TC_R, w), lambda i: (i + off, 0))
    return pl.pallas_call(
        _tc_search_kernel,
        grid=(n // TC_R,),
        in_specs=[shifted(nact), shifted(nact), shifted(nact), shifted(1)],
        out_specs=pl.BlockSpec((TC_R, 1), lambda i: (i, 0)),
        out_shape=jax.ShapeDtypeStruct((n, 1), jnp.float32),
    )(q1, q2, prop, split2)


def kernel(context, log_pi, propensity, split, W1a, b1a, W2a, b2a, W1b, b1b, W2b, b2b):
    del log_pi
    batch = context.shape[0]
    q1, q2 = _run_mlps(context, W1a, b1a, W2a, b2a, W1b, b1b, W2b, b2b,
                       0, batch)
    v1, v2 = _sc_quantiles(q1, q2, propensity, S_SC)
    split2 = split.reshape(batch, 1)
    b_tc = _tc_search(q1, q2, propensity, split2, S_SC)
    s_lo = split[:S_SC]
    b_sc = (1.0 - s_lo) * v1 + s_lo * v2
    return jnp.concatenate([b_sc, b_tc.reshape(batch - S_SC)])
